# Initial kernel scaffold; baseline (speedup 1.0000x reference)
import functools

import jax
import jax.numpy as jnp
from jax import lax
from jax.experimental import pallas as pl
from jax.experimental.pallas import tpu as pltpu
from jax.experimental.pallas import tpu_sc as plsc

_N, _E, _D = 10000, 320000, 128
_NC, _NS = 2, 16
_NW = _NC * _NS
_EPW = _E // _NW
_K = 80
_NB = _EPW // _K
_RPS = _N // _NS
_ZR = 125


def _sc_gather_segsum(x, snd3, rcv3):
    mesh = plsc.VectorSubcoreMesh(core_axis_name="c", subcore_axis_name="s")

    @functools.partial(
        pl.kernel,
        out_type=jax.ShapeDtypeStruct((_NC, _N, _D), jnp.float32),
        mesh=mesh,
        scratch_types=[
            pltpu.VMEM((_NB, _K), jnp.int32),
            pltpu.VMEM((_NB, _K), jnp.int32),
            pltpu.VMEM((_K, _D), jnp.float32),
            pltpu.VMEM((_K, _D), jnp.float32),
            pltpu.VMEM((_ZR, _D), jnp.float32),
            pltpu.VMEM_SHARED((_N, _D), jnp.float32),
            pltpu.SemaphoreType.DMA,
            pltpu.SemaphoreType.DMA,
        ],
    )
    def k(x_hbm, snd_hbm, rcv_hbm, out_hbm,
          snd_v, rcv_v, rows_a, rows_b, zeros_v, agg_sh, sem_a, sem_b):
        c = lax.axis_index("c")
        s = lax.axis_index("s")
        wid = s * _NC + c

        @pl.loop(0, _ZR)
        def _(r):
            @pl.loop(0, _D, step=16)
            def _(cc):
                zeros_v[r, pl.ds(cc, 16)] = jnp.zeros((16,), jnp.float32)

        @pl.loop(0, _RPS // _ZR)
        def _(z):
            pltpu.sync_copy(zeros_v, agg_sh.at[pl.ds(s * _RPS + z * _ZR, _ZR)])

        plsc.subcore_barrier()

        pltpu.sync_copy(snd_hbm.at[wid], snd_v)
        pltpu.sync_copy(rcv_hbm.at[wid], rcv_v)

        def scat(j, rows):
            pltpu.sync_copy(rows, agg_sh.at[rcv_v.at[j]], add=True)

        @pl.loop(0, _NB // 2)
        def _(p):
            j = 2 * p
            da = pltpu.async_copy(x_hbm.at[snd_v.at[j]], rows_a, sem_a)
            db = pltpu.async_copy(x_hbm.at[snd_v.at[j + 1]], rows_b, sem_b)
            da.wait()
            scat(j, rows_a)
            db.wait()
            scat(j + 1, rows_b)

        dl = pltpu.async_copy(x_hbm.at[snd_v.at[_NB - 1]], rows_a, sem_a)
        dl.wait()
        scat(_NB - 1, rows_a)

        plsc.subcore_barrier()

        @pl.loop(0, _RPS // _ZR)
        def _(z):
            base = s * _RPS + z * _ZR
            pltpu.sync_copy(agg_sh.at[pl.ds(base, _ZR)],
                            out_hbm.at[c, pl.ds(base, _ZR)])

    return k(x, snd3, rcv3)


def _dense_proj_body(agg_ref, x_ref, w_ref, b_ref, r_ref, rb_ref, o_ref):
    a = agg_ref[0] + agg_ref[1]
    h = jnp.dot(a, w_ref[...], preferred_element_type=jnp.float32,
                precision=lax.Precision.HIGHEST)
    h = jnp.maximum(h + b_ref[...], 0.0)
    res = jnp.dot(x_ref[...], r_ref[...], preferred_element_type=jnp.float32,
                  precision=lax.Precision.HIGHEST)
    o_ref[...] = h + res + rb_ref[...]


def _dense_id_body(agg_ref, x_ref, w_ref, b_ref, o_ref):
    a = agg_ref[0] + agg_ref[1]
    h = jnp.dot(a, w_ref[...], preferred_element_type=jnp.float32,
                precision=lax.Precision.HIGHEST)
    o_ref[...] = jnp.maximum(h + b_ref[...], 0.0) + x_ref[...]


_GB = 8
_BM = _N // _GB


def _dense_proj(agg2, x, w, b, r, rb):
    return pl.pallas_call(
        _dense_proj_body,
        out_shape=jax.ShapeDtypeStruct((_N, _D), jnp.float32),
        grid=(_GB,),
        in_specs=[
            pl.BlockSpec((_NC, _BM, _D), lambda i: (0, i, 0)),
            pl.BlockSpec((_BM, _D), lambda i: (i, 0)),
            pl.BlockSpec((_D, _D), lambda i: (0, 0)),
            pl.BlockSpec((1, _D), lambda i: (0, 0)),
            pl.BlockSpec((_D, _D), lambda i: (0, 0)),
            pl.BlockSpec((1, _D), lambda i: (0, 0)),
        ],
        out_specs=pl.BlockSpec((_BM, _D), lambda i: (i, 0)),
    )(agg2, x, w, b, r, rb)


def _dense_id(agg2, x, w, b):
    return pl.pallas_call(
        _dense_id_body,
        out_shape=jax.ShapeDtypeStruct((_N, _D), jnp.float32),
        grid=(_GB,),
        in_specs=[
            pl.BlockSpec((_NC, _BM, _D), lambda i: (0, i, 0)),
            pl.BlockSpec((_BM, _D), lambda i: (i, 0)),
            pl.BlockSpec((_D, _D), lambda i: (0, 0)),
            pl.BlockSpec((1, _D), lambda i: (0, 0)),
        ],
        out_specs=pl.BlockSpec((_BM, _D), lambda i: (i, 0)),
    )(agg2, x, w, b)


def kernel(x, senders, receivers, W0, b0, W1, b1, W2, b2, R0, rb0, R2, rb2):
    snd3 = senders.reshape(_NW, _NB, _K)
    rcv3 = receivers.reshape(_NW, _NB, _K)
    b0r, b1r, b2r = b0.reshape(1, _D), b1.reshape(1, _D), b2.reshape(1, _D)
    rb0r, rb2r = rb0.reshape(1, _D), rb2.reshape(1, _D)

    agg = _sc_gather_segsum(x, snd3, rcv3)
    x1 = _dense_proj(agg, x, W0, b0r, R0, rb0r)
    agg = _sc_gather_segsum(x1, snd3, rcv3)
    x2 = _dense_id(agg, x1, W1, b1r)
    agg = _sc_gather_segsum(x2, snd3, rcv3)
    return _dense_proj(agg, x2, W2, b2r, R2, rb2r)


# speedup vs baseline: 8.6073x; 8.6073x over previous
import functools

import jax
import jax.numpy as jnp
from jax import lax
from jax.experimental import pallas as pl
from jax.experimental.pallas import tpu as pltpu
from jax.experimental.pallas import tpu_sc as plsc

_N, _E, _D = 10000, 320000, 128
_NC, _NS = 2, 16
_NW = _NC * _NS
_EPW = _E // _NW
_K = 80
_NB = _EPW // _K
_NBC = 25
_NCH = _NB // _NBC
_ZR = 80
_NZC = _N // _ZR
_ZQ = -(-_NZC // _NS)


def _sc_gather_segsum(x, snd3, rcv3):
    mesh = plsc.VectorSubcoreMesh(core_axis_name="c", subcore_axis_name="s")

    @functools.partial(
        pl.kernel,
        out_type=jax.ShapeDtypeStruct((_NC, _N, _D), jnp.float32),
        mesh=mesh,
        scratch_types=[
            pltpu.VMEM((_NBC, _K), jnp.int32),
            pltpu.VMEM((_NBC, _K), jnp.int32),
            pltpu.VMEM((_K, _D), jnp.float32),
            pltpu.VMEM((_K, _D), jnp.float32),
            pltpu.VMEM_SHARED((_N, _D), jnp.float32),
            pltpu.SemaphoreType.DMA,
            pltpu.SemaphoreType.DMA,
        ],
    )
    def k(x_hbm, snd_hbm, rcv_hbm, out_hbm,
          snd_v, rcv_v, rows_a, rows_b, agg_sh, sem_a, sem_b):
        c = lax.axis_index("c")
        s = lax.axis_index("s")
        wid = s * _NC + c

        @pl.loop(0, _ZR)
        def _(r):
            @pl.loop(0, _D, step=16)
            def _(cc):
                rows_a[r, pl.ds(cc, 16)] = jnp.zeros((16,), jnp.float32)

        @pl.loop(0, _ZQ)
        def _(q):
            t = q * _NS + s

            @pl.when(t < _NZC)
            def _():
                pltpu.sync_copy(rows_a, agg_sh.at[pl.ds(t * _ZR, _ZR)])

        plsc.subcore_barrier()

        def scat(j, rows):
            pltpu.sync_copy(rows, agg_sh.at[rcv_v.at[j]], add=True)

        @pl.loop(0, _NCH)
        def _(h):
            pltpu.sync_copy(snd_hbm.at[wid, h], snd_v)
            pltpu.sync_copy(rcv_hbm.at[wid, h], rcv_v)

            @pl.loop(0, _NBC // 2)
            def _(p):
                j = 2 * p
                da = pltpu.async_copy(x_hbm.at[snd_v.at[j]], rows_a, sem_a)
                db = pltpu.async_copy(x_hbm.at[snd_v.at[j + 1]], rows_b, sem_b)
                da.wait()
                scat(j, rows_a)
                db.wait()
                scat(j + 1, rows_b)

            dl = pltpu.async_copy(x_hbm.at[snd_v.at[_NBC - 1]], rows_a, sem_a)
            dl.wait()
            scat(_NBC - 1, rows_a)

        plsc.subcore_barrier()

        @pl.loop(0, _ZQ)
        def _(q):
            t = q * _NS + s

            @pl.when(t < _NZC)
            def _():
                pltpu.sync_copy(agg_sh.at[pl.ds(t * _ZR, _ZR)],
                                out_hbm.at[c, pl.ds(t * _ZR, _ZR)])

    return k(x, snd3, rcv3)


def _dense_proj_body(agg_ref, x_ref, w_ref, b_ref, r_ref, rb_ref, o_ref):
    a = agg_ref[0] + agg_ref[1]
    h = jnp.dot(a, w_ref[...], preferred_element_type=jnp.float32,
                precision=lax.Precision.HIGHEST)
    h = jnp.maximum(h + b_ref[...], 0.0)
    res = jnp.dot(x_ref[...], r_ref[...], preferred_element_type=jnp.float32,
                  precision=lax.Precision.HIGHEST)
    o_ref[...] = h + res + rb_ref[...]


def _dense_id_body(agg_ref, x_ref, w_ref, b_ref, o_ref):
    a = agg_ref[0] + agg_ref[1]
    h = jnp.dot(a, w_ref[...], preferred_element_type=jnp.float32,
                precision=lax.Precision.HIGHEST)
    o_ref[...] = jnp.maximum(h + b_ref[...], 0.0) + x_ref[...]


_GB = 10
_BM = _N // _GB


def _dense_proj(agg2, x, w, b, r, rb):
    return pl.pallas_call(
        _dense_proj_body,
        out_shape=jax.ShapeDtypeStruct((_N, _D), jnp.float32),
        grid=(_GB,),
        in_specs=[
            pl.BlockSpec((_NC, _BM, _D), lambda i: (0, i, 0)),
            pl.BlockSpec((_BM, _D), lambda i: (i, 0)),
            pl.BlockSpec((_D, _D), lambda i: (0, 0)),
            pl.BlockSpec((1, _D), lambda i: (0, 0)),
            pl.BlockSpec((_D, _D), lambda i: (0, 0)),
            pl.BlockSpec((1, _D), lambda i: (0, 0)),
        ],
        out_specs=pl.BlockSpec((_BM, _D), lambda i: (i, 0)),
    )(agg2, x, w, b, r, rb)


def _dense_id(agg2, x, w, b):
    return pl.pallas_call(
        _dense_id_body,
        out_shape=jax.ShapeDtypeStruct((_N, _D), jnp.float32),
        grid=(_GB,),
        in_specs=[
            pl.BlockSpec((_NC, _BM, _D), lambda i: (0, i, 0)),
            pl.BlockSpec((_BM, _D), lambda i: (i, 0)),
            pl.BlockSpec((_D, _D), lambda i: (0, 0)),
            pl.BlockSpec((1, _D), lambda i: (0, 0)),
        ],
        out_specs=pl.BlockSpec((_BM, _D), lambda i: (i, 0)),
    )(agg2, x, w, b)


def kernel(x, senders, receivers, W0, b0, W1, b1, W2, b2, R0, rb0, R2, rb2):
    snd3 = senders.reshape(_NW, _NCH, _NBC, _K)
    rcv3 = receivers.reshape(_NW, _NCH, _NBC, _K)
    b0r, b1r, b2r = b0.reshape(1, _D), b1.reshape(1, _D), b2.reshape(1, _D)
    rb0r, rb2r = rb0.reshape(1, _D), rb2.reshape(1, _D)

    agg = _sc_gather_segsum(x, snd3, rcv3)
    x1 = _dense_proj(agg, x, W0, b0r, R0, rb0r)
    agg = _sc_gather_segsum(x1, snd3, rcv3)
    x2 = _dense_id(agg, x1, W1, b1r)
    agg = _sc_gather_segsum(x2, snd3, rcv3)
    return _dense_proj(agg, x2, W2, b2r, R2, rb2r)


# speedup vs baseline: 12.0276x; 1.3974x over previous
import functools

import jax
import jax.numpy as jnp
from jax import lax
from jax.experimental import pallas as pl
from jax.experimental.pallas import tpu as pltpu
from jax.experimental.pallas import tpu_sc as plsc

_N, _E, _D = 10000, 320000, 128
_NC, _NS = 2, 16
_NW = _NC * _NS
_EPW = _E // _NW
_K = 80
_NB = _EPW // _K
_NBC = 25
_NCH = _NB // _NBC
_ZR = 80
_NZC = _N // _ZR
_ZQ = -(-_NZC // _NS)


def _sc_gather_segsum(x, snd3, rcv3):
    mesh = plsc.VectorSubcoreMesh(core_axis_name="c", subcore_axis_name="s")

    @functools.partial(
        pl.kernel,
        out_type=jax.ShapeDtypeStruct((_NC, _N, _D), jnp.float32),
        mesh=mesh,
        scratch_types=[
            pltpu.VMEM((_NBC, _K), jnp.int32),
            pltpu.VMEM((_NBC, _K), jnp.int32),
            pltpu.VMEM((3, _K, _D), jnp.float32),
            pltpu.VMEM_SHARED((_N, _D), jnp.float32),
            pltpu.SemaphoreType.DMA((3,)),
            pltpu.SemaphoreType.DMA((3,)),
        ],
    )
    def k(x_hbm, snd_hbm, rcv_hbm, out_hbm,
          snd_v, rcv_v, bufs, agg_sh, gsem, ssem):
        c = lax.axis_index("c")
        s = lax.axis_index("s")
        wid = s * _NC + c

        @pl.loop(0, _ZR)
        def _(r):
            @pl.loop(0, _D, step=16)
            def _(cc):
                bufs[0, r, pl.ds(cc, 16)] = jnp.zeros((16,), jnp.float32)

        @pl.loop(0, _ZQ)
        def _(q):
            t = q * _NS + s

            @pl.when(t < _NZC)
            def _():
                pltpu.sync_copy(bufs.at[0], agg_sh.at[pl.ds(t * _ZR, _ZR)])

        plsc.subcore_barrier()

        def g_start(j, b):
            pltpu.async_copy(x_hbm.at[snd_v.at[j]], bufs.at[b], gsem.at[b])

        def g_wait(j, b):
            pltpu.make_async_copy(
                x_hbm.at[snd_v.at[j]], bufs.at[b], gsem.at[b]).wait()

        def s_start(j, b):
            pltpu.async_copy(bufs.at[b], agg_sh.at[rcv_v.at[j]], ssem.at[b],
                             add=True)

        def s_wait(j, b):
            pltpu.make_async_copy(
                bufs.at[b], agg_sh.at[rcv_v.at[j]], ssem.at[b]).wait()

        @pl.loop(0, _NCH)
        def _(h):
            pltpu.sync_copy(snd_hbm.at[wid, h], snd_v)
            pltpu.sync_copy(rcv_hbm.at[wid, h], rcv_v)

            @pl.loop(0, _NBC + 2)
            def _(p):
                @pl.when(p < _NBC)
                def _():
                    b = lax.rem(p, 3)

                    @pl.when(p >= 3)
                    def _():
                        s_wait(p - 3, b)

                    g_start(p, b)

                @pl.when(p >= 2)
                def _():
                    q = p - 2
                    bq = lax.rem(q, 3)
                    g_wait(q, bq)
                    s_start(q, bq)

            @pl.loop(_NBC - 3, _NBC)
            def _(j):
                s_wait(j, lax.rem(j, 3))

        plsc.subcore_barrier()

        @pl.loop(0, _ZQ)
        def _(q):
            t = q * _NS + s

            @pl.when(t < _NZC)
            def _():
                pltpu.sync_copy(agg_sh.at[pl.ds(t * _ZR, _ZR)],
                                out_hbm.at[c, pl.ds(t * _ZR, _ZR)])

    return k(x, snd3, rcv3)


def _dense_proj_body(agg_ref, x_ref, w_ref, b_ref, r_ref, rb_ref, o_ref):
    a = agg_ref[0] + agg_ref[1]
    h = jnp.dot(a, w_ref[...], preferred_element_type=jnp.float32,
                precision=lax.Precision.HIGHEST)
    h = jnp.maximum(h + b_ref[...], 0.0)
    res = jnp.dot(x_ref[...], r_ref[...], preferred_element_type=jnp.float32,
                  precision=lax.Precision.HIGHEST)
    o_ref[...] = h + res + rb_ref[...]


def _dense_id_body(agg_ref, x_ref, w_ref, b_ref, o_ref):
    a = agg_ref[0] + agg_ref[1]
    h = jnp.dot(a, w_ref[...], preferred_element_type=jnp.float32,
                precision=lax.Precision.HIGHEST)
    o_ref[...] = jnp.maximum(h + b_ref[...], 0.0) + x_ref[...]


_GB = 10
_BM = _N // _GB


def _dense_proj(agg2, x, w, b, r, rb):
    return pl.pallas_call(
        _dense_proj_body,
        out_shape=jax.ShapeDtypeStruct((_N, _D), jnp.float32),
        grid=(_GB,),
        in_specs=[
            pl.BlockSpec((_NC, _BM, _D), lambda i: (0, i, 0)),
            pl.BlockSpec((_BM, _D), lambda i: (i, 0)),
            pl.BlockSpec((_D, _D), lambda i: (0, 0)),
            pl.BlockSpec((1, _D), lambda i: (0, 0)),
            pl.BlockSpec((_D, _D), lambda i: (0, 0)),
            pl.BlockSpec((1, _D), lambda i: (0, 0)),
        ],
        out_specs=pl.BlockSpec((_BM, _D), lambda i: (i, 0)),
    )(agg2, x, w, b, r, rb)


def _dense_id(agg2, x, w, b):
    return pl.pallas_call(
        _dense_id_body,
        out_shape=jax.ShapeDtypeStruct((_N, _D), jnp.float32),
        grid=(_GB,),
        in_specs=[
            pl.BlockSpec((_NC, _BM, _D), lambda i: (0, i, 0)),
            pl.BlockSpec((_BM, _D), lambda i: (i, 0)),
            pl.BlockSpec((_D, _D), lambda i: (0, 0)),
            pl.BlockSpec((1, _D), lambda i: (0, 0)),
        ],
        out_specs=pl.BlockSpec((_BM, _D), lambda i: (i, 0)),
    )(agg2, x, w, b)


def kernel(x, senders, receivers, W0, b0, W1, b1, W2, b2, R0, rb0, R2, rb2):
    snd3 = senders.reshape(_NW, _NCH, _NBC, _K)
    rcv3 = receivers.reshape(_NW, _NCH, _NBC, _K)
    b0r, b1r, b2r = b0.reshape(1, _D), b1.reshape(1, _D), b2.reshape(1, _D)
    rb0r, rb2r = rb0.reshape(1, _D), rb2.reshape(1, _D)

    agg = _sc_gather_segsum(x, snd3, rcv3)
    x1 = _dense_proj(agg, x, W0, b0r, R0, rb0r)
    agg = _sc_gather_segsum(x1, snd3, rcv3)
    x2 = _dense_id(agg, x1, W1, b1r)
    agg = _sc_gather_segsum(x2, snd3, rcv3)
    return _dense_proj(agg, x2, W2, b2r, R2, rb2r)
